# baseline (speedup 1.0000x reference)
import jax
import jax.numpy as jnp
from jax.experimental import pallas as pl
from jax.experimental.pallas import tpu as pltpu

EPS_DICE = 0.0001


def _dice_block_kernel(out_ref, tgt_ref, acc_ref):
    c, h, w = out_ref.shape[1:]
    hs = 8
    cls = jax.lax.broadcasted_iota(jnp.int32, (c, hs, w), 0)
    acc_tot = jnp.zeros((c, hs, w), jnp.float32)
    acc_int = jnp.zeros((c, hs, w), jnp.float32)
    acc_cnt = jnp.zeros((c, hs, w), jnp.float32)

    def body(k, accs):
        a_tot, a_int, a_cnt = accs
        xk = out_ref[0, :, pl.ds(k * hs, hs), :]
        tk = tgt_ref[0, pl.ds(k * hs, hs), :]
        th = jnp.tanh(xk * 0.5)
        m = tk[None, :, :] == cls
        a_tot = a_tot + th
        a_int = a_int + jnp.where(m, th, 0.0)
        a_cnt = a_cnt + jnp.where(m, 1.0, 0.0)
        return a_tot, a_int, a_cnt

    acc_tot, acc_int, acc_cnt = jax.lax.fori_loop(
        0, h // hs, body, (acc_tot, acc_int, acc_cnt)
    )
    t_tot = jnp.sum(acc_tot, axis=(1, 2))
    t_int = jnp.sum(acc_int, axis=(1, 2))
    cnt = jnp.sum(acc_cnt, axis=(1, 2))
    acc_ref[0, 0] = jnp.concatenate([t_tot, t_int, cnt])


@jax.jit
def kernel(output, target):
    b, c, h, w = output.shape
    tgt = target.astype(jnp.int32)
    acc = pl.pallas_call(
        _dice_block_kernel,
        grid=(b,),
        in_specs=[
            pl.BlockSpec((1, c, h, w), lambda i: (i, 0, 0, 0)),
            pl.BlockSpec((1, h, w), lambda i: (i, 0, 0)),
        ],
        out_specs=pl.BlockSpec((1, 1, 3 * c), lambda i: (i, 0, 0)),
        out_shape=jax.ShapeDtypeStruct((b, 1, 3 * c), jnp.float32),
        compiler_params=pltpu.CompilerParams(
            dimension_semantics=("arbitrary",),
        ),
    )(output, tgt)
    t_tot = acc[:, 0, :c]
    t_int = acc[:, 0, c : 2 * c]
    cnt = acc[:, 0, 2 * c :]
    hw = jnp.float32(h * w)
    sig_sum = 0.5 * t_tot + 0.5 * hw
    inter = 0.5 * t_int + 0.5 * cnt
    loss_per_channel = 1.0 - 2.0 * inter / (sig_sum + cnt + EPS_DICE)
    return loss_per_channel.sum(axis=1) / c


# speedup vs baseline: 1.0550x; 1.0550x over previous
import jax
import jax.numpy as jnp
from jax.experimental import pallas as pl
from jax.experimental.pallas import tpu as pltpu

EPS_DICE = 0.0001


def _dice_block_kernel(out_ref, tgt_ref, acc_ref):
    x = out_ref[0]
    t = tgt_ref[0]
    cls = jax.lax.broadcasted_iota(jnp.int32, x.shape, 0).astype(jnp.float32)
    th = jnp.tanh(x * 0.5)
    m = t[None, :, :] == cls
    t_tot = jnp.sum(th, axis=(1, 2))
    t_int = jnp.sum(jnp.where(m, th, 0.0), axis=(1, 2))
    cnt = jnp.sum(jnp.where(m, 1.0, 0.0), axis=(1, 2))
    acc_ref[0, 0, 0] = jnp.concatenate([t_tot, t_int, cnt])


@jax.jit
def kernel(output, target):
    b, c, h, w = output.shape
    hsplit = 2
    hb = h // hsplit
    tgt = target.astype(jnp.float32)
    acc = pl.pallas_call(
        _dice_block_kernel,
        grid=(b, hsplit),
        in_specs=[
            pl.BlockSpec((1, c, hb, w), lambda i, j: (i, 0, j, 0)),
            pl.BlockSpec((1, hb, w), lambda i, j: (i, j, 0)),
        ],
        out_specs=pl.BlockSpec((1, 1, 1, 3 * c), lambda i, j: (i, j, 0, 0)),
        out_shape=jax.ShapeDtypeStruct((b, hsplit, 1, 3 * c), jnp.float32),
        compiler_params=pltpu.CompilerParams(
            dimension_semantics=("arbitrary", "arbitrary"),
        ),
    )(output, tgt)
    acc = acc[:, :, 0, :].sum(axis=1)
    t_tot = acc[:, :c]
    t_int = acc[:, c : 2 * c]
    cnt = acc[:, 2 * c :]
    hw = jnp.float32(h * w)
    sig_sum = 0.5 * t_tot + 0.5 * hw
    inter = 0.5 * t_int + 0.5 * cnt
    loss_per_channel = 1.0 - 2.0 * inter / (sig_sum + cnt + EPS_DICE)
    return loss_per_channel.sum(axis=1) / c
